# baseline (speedup 1.0000x reference)
import functools

import jax
import jax.numpy as jnp
from jax import lax
from jax.experimental import pallas as pl
from jax.experimental.pallas import tpu as pltpu
from jax.experimental.pallas import tpu_sc as plsc

NUM_CODEBOOKS = 6
VOCAB = 1000
EMB_DIM = 1024
BATCH = 8
TIME = 1024
TOTAL_ROWS = BATCH * TIME * NUM_CODEBOOKS
TABLE_ROWS = NUM_CODEBOOKS * VOCAB

_NC, _NS, _LANES = 2, 16, 16
_NW = _NC * _NS
_ROWS_PER_W = TOTAL_ROWS // _NW
_CHUNK = 24
_NCHUNK = _ROWS_PER_W // _CHUNK
_NBUF = 4


_PREP_BLK = 200


def _prep_body(tab_ref, pe_ref, out_ref):
    c = pl.program_id(0) // (VOCAB // _PREP_BLK)
    val = tab_ref[...] + pe_ref[pl.ds(c, 1), :]
    out_ref[...] = val.reshape(_PREP_BLK * EMB_DIM)


def _make_table_pe(table, pe6):
    return pl.pallas_call(
        _prep_body,
        grid=(TABLE_ROWS // _PREP_BLK,),
        in_specs=[
            pl.BlockSpec((_PREP_BLK, EMB_DIM), lambda i: (i, 0)),
            pl.BlockSpec((NUM_CODEBOOKS, EMB_DIM), lambda i: (0, 0)),
        ],
        out_specs=pl.BlockSpec((_PREP_BLK * EMB_DIM,), lambda i: (i,)),
        out_shape=jax.ShapeDtypeStruct((TABLE_ROWS * EMB_DIM,), jnp.float32),
    )(table, pe6)


def _sc_pipeline(tok_hbm, tpe_hbm, out_hbm, tok_v, idx_v, bufs, gsems, ssems):
    wid = lax.axis_index("s") * _NC + lax.axis_index("c")
    base = wid * _ROWS_PER_W
    l0 = (wid % 4) * _ROWS_PER_W
    pltpu.sync_copy(tok_hbm.at[pl.ds(pl.multiple_of(base, 8), _ROWS_PER_W)], tok_v)
    for j in range(_ROWS_PER_W // _LANES):
        sl = pl.ds(_LANES * j, _LANES)
        l = lax.iota(jnp.int32, _LANES) + (l0 + _LANES * j)
        idx_v[sl] = tok_v[sl] + lax.shift_right_logical(l, 10) * VOCAB

    def fire_gather(g, rows_v, sem):
        pltpu.async_copy(tpe_hbm.at[idx_v.at[pl.ds(g * _CHUNK, _CHUNK)]], rows_v, sem)

    def wait_gather(g, rows_v, sem):
        pltpu.make_async_copy(
            tpe_hbm.at[idx_v.at[pl.ds(g * _CHUNK, _CHUNK)]], rows_v, sem
        ).wait()

    def out_slice(g):
        return out_hbm.at[pl.ds(pl.multiple_of(base + g * _CHUNK, 8), _CHUNK)]

    def fire_store(g, rows_v, sem):
        pltpu.async_copy(rows_v.reshape(_CHUNK, EMB_DIM), out_slice(g), sem)

    def wait_store(g, rows_v, sem):
        pltpu.make_async_copy(rows_v.reshape(_CHUNK, EMB_DIM), out_slice(g), sem).wait()

    fire_gather(0, bufs[0], gsems[0])
    fire_gather(1, bufs[1], gsems[1])

    def quad_body(h, carry):
        for k in range(_NBUF):
            g = _NBUF * h + k
            b = k
            b2 = (k + 2) % _NBUF
            wait_gather(g, bufs[b], gsems[b])
            fire_store(g, bufs[b], ssems[b])

            @pl.when(g - 2 >= 0)
            def _():
                wait_store(g - 2, bufs[b2], ssems[b2])

            @pl.when(g + 2 < _NCHUNK)
            def _():
                fire_gather(g + 2, bufs[b2], gsems[b2])

        return carry

    lax.fori_loop(0, _NCHUNK // _NBUF, quad_body, 0)
    wait_store(_NCHUNK - 2, bufs[(_NCHUNK - 2) % _NBUF], ssems[(_NCHUNK - 2) % _NBUF])
    wait_store(_NCHUNK - 1, bufs[(_NCHUNK - 1) % _NBUF], ssems[(_NCHUNK - 1) % _NBUF])


def _sc_body(tok_hbm, tpe_hbm, out_hbm, tok_v, idx_v,
             rows0, rows1, rows2, rows3,
             gsem0, gsem1, gsem2, gsem3, ssem0, ssem1, ssem2, ssem3):
    _sc_pipeline(
        tok_hbm, tpe_hbm, out_hbm, tok_v, idx_v,
        [rows0, rows1, rows2, rows3],
        [gsem0, gsem1, gsem2, gsem3],
        [ssem0, ssem1, ssem2, ssem3],
    )


def _sc_gather(tok_flat, table_pe):
    mesh = plsc.VectorSubcoreMesh(core_axis_name="c", subcore_axis_name="s")
    return pl.kernel(
        _sc_body,
        out_type=jax.ShapeDtypeStruct((TOTAL_ROWS, EMB_DIM), jnp.float32),
        mesh=mesh,
        scratch_types=[
            pltpu.VMEM((_ROWS_PER_W,), jnp.int32),
            pltpu.VMEM((_ROWS_PER_W,), jnp.int32),
        ]
        + [pltpu.VMEM((_CHUNK, 8, EMB_DIM // 8), jnp.float32)] * _NBUF
        + [pltpu.SemaphoreType.DMA] * (2 * _NBUF),
    )(tok_flat, table_pe)


def kernel(in_tokens, table, pos_encoding):
    pe6 = pos_encoding.reshape(NUM_CODEBOOKS, EMB_DIM)
    table_pe = _make_table_pe(table, pe6).reshape(TABLE_ROWS, 8, EMB_DIM // 8)
    tok_flat = jnp.transpose(in_tokens, (0, 2, 1)).reshape(TOTAL_ROWS)
    out_flat = _sc_gather(tok_flat, table_pe)
    out_bct = out_flat.reshape(BATCH, NUM_CODEBOOKS, TIME, EMB_DIM)
    return jnp.transpose(out_bct, (0, 2, 1, 3))


# speedup vs baseline: 1.0326x; 1.0326x over previous
import functools

import jax
import jax.numpy as jnp
from jax import lax
from jax.experimental import pallas as pl
from jax.experimental.pallas import tpu as pltpu
from jax.experimental.pallas import tpu_sc as plsc

NUM_CODEBOOKS = 6
VOCAB = 1000
EMB_DIM = 1024
BATCH = 8
TIME = 1024
TOTAL_ROWS = BATCH * TIME * NUM_CODEBOOKS
TABLE_ROWS = NUM_CODEBOOKS * VOCAB

_NC, _NS, _LANES = 2, 16, 16
_NW = _NC * _NS
_ROWS_PER_W = TOTAL_ROWS // _NW
_CHUNK = 32
_NCHUNK = _ROWS_PER_W // _CHUNK
_NBUF = 3


def _sc_pipeline(tok_hbm, tab_hbm, pe_hbm, out_hbm, tok_v, idx_v, pe_v,
                 bufs, gsems, ssems):
    wid = lax.axis_index("s") * _NC + lax.axis_index("c")
    base = wid * _ROWS_PER_W
    pltpu.sync_copy(pe_hbm, pe_v)
    pltpu.sync_copy(tok_hbm.at[pl.ds(pl.multiple_of(base, 8), _ROWS_PER_W)], tok_v)
    for j in range(_ROWS_PER_W // _LANES):
        sl = pl.ds(_LANES * j, _LANES)
        r = lax.iota(jnp.int32, _LANES) + (base + _LANES * j)
        c = lax.rem(lax.shift_right_logical(r, 10), NUM_CODEBOOKS)
        idx_v[sl] = tok_v[sl] + c * VOCAB

    def fire_gather(g, rows_v, sem):
        pltpu.async_copy(tab_hbm.at[idx_v.at[pl.ds(g * _CHUNK, _CHUNK)]], rows_v, sem)

    def wait_gather(g, rows_v, sem):
        pltpu.make_async_copy(
            tab_hbm.at[idx_v.at[pl.ds(g * _CHUNK, _CHUNK)]], rows_v, sem
        ).wait()

    def out_slice(g):
        return out_hbm.at[pl.ds(pl.multiple_of(base + g * _CHUNK, 8), _CHUNK)]

    def fire_store(g, rows_v, sem):
        pltpu.async_copy(rows_v, out_slice(g), sem)

    def wait_store(g, rows_v, sem):
        pltpu.make_async_copy(rows_v, out_slice(g), sem).wait()

    def add_pe(g, rows_v):
        c = lax.rem(lax.shift_right_logical(base + g * _CHUNK, 10), NUM_CODEBOOKS)

        def dcol(d, carry):
            pe_slice = pe_v[c, pl.ds(d * _LANES, _LANES)]
            for i in range(_CHUNK):
                sl = pl.ds(d * _LANES, _LANES)
                rows_v[i, sl] = rows_v[i, sl] + pe_slice
            return carry

        lax.fori_loop(0, EMB_DIM // _LANES, dcol, 0)

    fire_gather(0, bufs[0], gsems[0])
    fire_gather(1, bufs[1], gsems[1])

    def ring_body(h, carry):
        for k in range(_NBUF):
            g = _NBUF * h + k
            b2 = (k + 2) % _NBUF
            wait_gather(g, bufs[k], gsems[k])
            gp = g + 2 - _NBUF

            @pl.when(gp >= 0)
            def _():
                wait_store(gp, bufs[b2], ssems[b2])

            @pl.when(g + 2 < _NCHUNK)
            def _():
                fire_gather(g + 2, bufs[b2], gsems[b2])

            add_pe(g, bufs[k])
            fire_store(g, bufs[k], ssems[k])

        return carry

    lax.fori_loop(0, _NCHUNK // _NBUF, ring_body, 0)
    for q in range(_NCHUNK + 2 - _NBUF, _NCHUNK):
        wait_store(q, bufs[q % _NBUF], ssems[q % _NBUF])


def _sc_body(tok_hbm, tab_hbm, pe_hbm, out_hbm, tok_v, idx_v, pe_v,
             rows0, rows1, rows2,
             gsem0, gsem1, gsem2, ssem0, ssem1, ssem2):
    _sc_pipeline(
        tok_hbm, tab_hbm, pe_hbm, out_hbm, tok_v, idx_v, pe_v,
        [rows0, rows1, rows2],
        [gsem0, gsem1, gsem2],
        [ssem0, ssem1, ssem2],
    )


def _sc_gather(tok_flat, table, pe6):
    mesh = plsc.VectorSubcoreMesh(core_axis_name="c", subcore_axis_name="s")
    return pl.kernel(
        _sc_body,
        out_type=jax.ShapeDtypeStruct((TOTAL_ROWS, EMB_DIM), jnp.float32),
        mesh=mesh,
        scratch_types=[
            pltpu.VMEM((_ROWS_PER_W,), jnp.int32),
            pltpu.VMEM((_ROWS_PER_W,), jnp.int32),
            pltpu.VMEM((NUM_CODEBOOKS, EMB_DIM), jnp.float32),
        ]
        + [pltpu.VMEM((_CHUNK, EMB_DIM), jnp.float32)] * _NBUF
        + [pltpu.SemaphoreType.DMA] * (2 * _NBUF),
    )(tok_flat, table, pe6)


def kernel(in_tokens, table, pos_encoding):
    pe6 = pos_encoding.reshape(NUM_CODEBOOKS, EMB_DIM)
    tok_flat = jnp.transpose(in_tokens, (0, 2, 1)).reshape(TOTAL_ROWS)
    out_flat = _sc_gather(tok_flat, table, pe6)
    out_bct = out_flat.reshape(BATCH, NUM_CODEBOOKS, TIME, EMB_DIM)
    return jnp.transpose(out_bct, (0, 2, 1, 3))


# speedup vs baseline: 1.0702x; 1.0364x over previous
import functools

import jax
import jax.numpy as jnp
from jax import lax
from jax.experimental import pallas as pl
from jax.experimental.pallas import tpu as pltpu
from jax.experimental.pallas import tpu_sc as plsc

NUM_CODEBOOKS = 6
VOCAB = 1000
EMB_DIM = 1024
BATCH = 8
TIME = 1024
TOTAL_ROWS = BATCH * TIME * NUM_CODEBOOKS
TABLE_ROWS = NUM_CODEBOOKS * VOCAB

_NC, _NS, _LANES = 2, 16, 16
_NW = _NC * _NS
_ROWS_PER_W = TOTAL_ROWS // _NW
_CHUNK = 24
_NCHUNK = _ROWS_PER_W // _CHUNK
_NBUF = 4


def _prep_body(tab_ref, pe_ref, out_ref):
    out_ref[...] = tab_ref[...] + pe_ref[pl.ds(pl.program_id(0), 1), :]


def _make_table_pe(table, pe6):
    blk = VOCAB
    return pl.pallas_call(
        _prep_body,
        grid=(TABLE_ROWS // blk,),
        in_specs=[
            pl.BlockSpec((blk, EMB_DIM), lambda i: (i, 0)),
            pl.BlockSpec((NUM_CODEBOOKS, EMB_DIM), lambda i: (0, 0)),
        ],
        out_specs=pl.BlockSpec((blk, EMB_DIM), lambda i: (i, 0)),
        out_shape=jax.ShapeDtypeStruct((TABLE_ROWS, EMB_DIM), jnp.float32),
    )(table, pe6)


def _sc_pipeline(tok_hbm, tpe_hbm, out_hbm, tok_v, idx_v, bufs, gsems, ssems):
    wid = lax.axis_index("s") * _NC + lax.axis_index("c")
    base = wid * _ROWS_PER_W
    l0 = (wid % 4) * _ROWS_PER_W
    pltpu.sync_copy(tok_hbm.at[pl.ds(pl.multiple_of(base, 8), _ROWS_PER_W)], tok_v)
    for j in range(_ROWS_PER_W // _LANES):
        sl = pl.ds(_LANES * j, _LANES)
        l = lax.iota(jnp.int32, _LANES) + (l0 + _LANES * j)
        idx_v[sl] = tok_v[sl] + lax.shift_right_logical(l, 10) * VOCAB

    def fire_gather(g, rows_v, sem):
        pltpu.async_copy(tpe_hbm.at[idx_v.at[pl.ds(g * _CHUNK, _CHUNK)]], rows_v, sem)

    def wait_gather(g, rows_v, sem):
        pltpu.make_async_copy(
            tpe_hbm.at[idx_v.at[pl.ds(g * _CHUNK, _CHUNK)]], rows_v, sem
        ).wait()

    def out_slice(g):
        return out_hbm.at[pl.ds(pl.multiple_of(base + g * _CHUNK, 8), _CHUNK)]

    def fire_store(g, rows_v, sem):
        pltpu.async_copy(rows_v, out_slice(g), sem)

    def wait_store(g, rows_v, sem):
        pltpu.make_async_copy(rows_v, out_slice(g), sem).wait()

    fire_gather(0, bufs[0], gsems[0])
    fire_gather(1, bufs[1], gsems[1])

    def quad_body(h, carry):
        for k in range(_NBUF):
            g = _NBUF * h + k
            b = k
            b2 = (k + 2) % _NBUF
            wait_gather(g, bufs[b], gsems[b])
            fire_store(g, bufs[b], ssems[b])

            @pl.when(g - 2 >= 0)
            def _():
                wait_store(g - 2, bufs[b2], ssems[b2])

            @pl.when(g + 2 < _NCHUNK)
            def _():
                fire_gather(g + 2, bufs[b2], gsems[b2])

        return carry

    lax.fori_loop(0, _NCHUNK // _NBUF, quad_body, 0)
    wait_store(_NCHUNK - 2, bufs[(_NCHUNK - 2) % _NBUF], ssems[(_NCHUNK - 2) % _NBUF])
    wait_store(_NCHUNK - 1, bufs[(_NCHUNK - 1) % _NBUF], ssems[(_NCHUNK - 1) % _NBUF])


def _sc_body(tok_hbm, tpe_hbm, out_hbm, tok_v, idx_v,
             rows0, rows1, rows2, rows3,
             gsem0, gsem1, gsem2, gsem3, ssem0, ssem1, ssem2, ssem3):
    _sc_pipeline(
        tok_hbm, tpe_hbm, out_hbm, tok_v, idx_v,
        [rows0, rows1, rows2, rows3],
        [gsem0, gsem1, gsem2, gsem3],
        [ssem0, ssem1, ssem2, ssem3],
    )


def _sc_gather(tok_flat, table_pe):
    mesh = plsc.VectorSubcoreMesh(core_axis_name="c", subcore_axis_name="s")
    return pl.kernel(
        _sc_body,
        out_type=jax.ShapeDtypeStruct((TOTAL_ROWS, EMB_DIM), jnp.float32),
        mesh=mesh,
        scratch_types=[
            pltpu.VMEM((_ROWS_PER_W,), jnp.int32),
            pltpu.VMEM((_ROWS_PER_W,), jnp.int32),
        ]
        + [pltpu.VMEM((_CHUNK, EMB_DIM), jnp.float32)] * _NBUF
        + [pltpu.SemaphoreType.DMA] * (2 * _NBUF),
    )(tok_flat, table_pe)


def kernel(in_tokens, table, pos_encoding):
    pe6 = pos_encoding.reshape(NUM_CODEBOOKS, EMB_DIM)
    table_pe = _make_table_pe(table, pe6)
    tok_flat = jnp.transpose(in_tokens, (0, 2, 1)).reshape(TOTAL_ROWS)
    out_flat = _sc_gather(tok_flat, table_pe)
    out_bct = out_flat.reshape(BATCH, NUM_CODEBOOKS, TIME, EMB_DIM)
    return jnp.transpose(out_bct, (0, 2, 1, 3))


# speedup vs baseline: 1.0786x; 1.0079x over previous
import jax
import jax.numpy as jnp
from jax import lax
from jax.experimental import pallas as pl
from jax.experimental.pallas import tpu as pltpu
from jax.experimental.pallas import tpu_sc as plsc

NUM_CODEBOOKS = 6
VOCAB = 1000
EMB_DIM = 1024
BATCH = 8
TIME = 1024
TOTAL_ROWS = BATCH * TIME * NUM_CODEBOOKS
TABLE_ROWS = NUM_CODEBOOKS * VOCAB

_NC, _NS, _LANES = 2, 16, 16
_NW = _NC * _NS
_ROWS_PER_W = TOTAL_ROWS // _NW
_CHUNK = 16
_NCHUNK = _ROWS_PER_W // _CHUNK
_NBUF = 6
_AHEAD = 3


def _prep_body(tab_ref, pe_ref, out_ref):
    out_ref[...] = tab_ref[...] + pe_ref[pl.ds(pl.program_id(0), 1), :]


def _make_table_pe(table, pe6):
    blk = VOCAB
    return pl.pallas_call(
        _prep_body,
        grid=(TABLE_ROWS // blk,),
        in_specs=[
            pl.BlockSpec((blk, EMB_DIM), lambda i: (i, 0)),
            pl.BlockSpec((NUM_CODEBOOKS, EMB_DIM), lambda i: (0, 0)),
        ],
        out_specs=pl.BlockSpec((blk, EMB_DIM), lambda i: (i, 0)),
        out_shape=jax.ShapeDtypeStruct((TABLE_ROWS, EMB_DIM), jnp.float32),
    )(table, pe6)


def _sc_pipeline(tok_hbm, tpe_hbm, out_hbm, tok_v, idx_v, bufs, gsems, ssems):
    wid = lax.axis_index("s") * _NC + lax.axis_index("c")
    base = wid * _ROWS_PER_W
    l0 = (wid % 4) * _ROWS_PER_W
    pltpu.sync_copy(tok_hbm.at[pl.ds(pl.multiple_of(base, 8), _ROWS_PER_W)], tok_v)
    for j in range(_ROWS_PER_W // _LANES):
        sl = pl.ds(_LANES * j, _LANES)
        l = lax.iota(jnp.int32, _LANES) + (l0 + _LANES * j)
        idx_v[sl] = tok_v[sl] + lax.shift_right_logical(l, 10) * VOCAB

    def fire_gather(g, rows_v, sem):
        pltpu.async_copy(tpe_hbm.at[idx_v.at[pl.ds(g * _CHUNK, _CHUNK)]], rows_v, sem)

    def wait_gather(g, rows_v, sem):
        pltpu.make_async_copy(
            tpe_hbm.at[idx_v.at[pl.ds(g * _CHUNK, _CHUNK)]], rows_v, sem
        ).wait()

    def out_slice(g):
        return out_hbm.at[pl.ds(pl.multiple_of(base + g * _CHUNK, 8), _CHUNK)]

    def fire_store(g, rows_v, sem):
        pltpu.async_copy(rows_v, out_slice(g), sem)

    def wait_store(g, rows_v, sem):
        pltpu.make_async_copy(rows_v, out_slice(g), sem).wait()

    for a in range(_AHEAD):
        fire_gather(a, bufs[a], gsems[a])

    def ring_body(h, carry):
        for k in range(_NBUF):
            g = _NBUF * h + k
            b2 = (k + _AHEAD) % _NBUF
            wait_gather(g, bufs[k], gsems[k])
            fire_store(g, bufs[k], ssems[k])
            gp = g + _AHEAD - _NBUF

            @pl.when(gp >= 0)
            def _():
                wait_store(gp, bufs[b2], ssems[b2])

            @pl.when(g + _AHEAD < _NCHUNK)
            def _():
                fire_gather(g + _AHEAD, bufs[b2], gsems[b2])

        return carry

    lax.fori_loop(0, _NCHUNK // _NBUF, ring_body, 0)
    for q in range(_NCHUNK + _AHEAD - _NBUF, _NCHUNK):
        wait_store(q, bufs[q % _NBUF], ssems[q % _NBUF])


def _sc_body(tok_hbm, tpe_hbm, out_hbm, tok_v, idx_v,
             rows0, rows1, rows2, rows3, rows4, rows5,
             gsem0, gsem1, gsem2, gsem3, gsem4, gsem5,
             ssem0, ssem1, ssem2, ssem3, ssem4, ssem5):
    _sc_pipeline(
        tok_hbm, tpe_hbm, out_hbm, tok_v, idx_v,
        [rows0, rows1, rows2, rows3, rows4, rows5],
        [gsem0, gsem1, gsem2, gsem3, gsem4, gsem5],
        [ssem0, ssem1, ssem2, ssem3, ssem4, ssem5],
    )


def _sc_gather(tok_flat, table_pe):
    mesh = plsc.VectorSubcoreMesh(core_axis_name="c", subcore_axis_name="s")
    return pl.kernel(
        _sc_body,
        out_type=jax.ShapeDtypeStruct((TOTAL_ROWS, EMB_DIM), jnp.float32),
        mesh=mesh,
        scratch_types=[
            pltpu.VMEM((_ROWS_PER_W,), jnp.int32),
            pltpu.VMEM((_ROWS_PER_W,), jnp.int32),
        ]
        + [pltpu.VMEM((_CHUNK, EMB_DIM), jnp.float32)] * _NBUF
        + [pltpu.SemaphoreType.DMA] * (2 * _NBUF),
    )(tok_flat, table_pe)


def kernel(in_tokens, table, pos_encoding):
    pe6 = pos_encoding.reshape(NUM_CODEBOOKS, EMB_DIM)
    table_pe = _make_table_pe(table, pe6)
    tok_flat = jnp.transpose(in_tokens, (0, 2, 1)).reshape(TOTAL_ROWS)
    out_flat = _sc_gather(tok_flat, table_pe)
    out_bct = out_flat.reshape(BATCH, NUM_CODEBOOKS, TIME, EMB_DIM)
    return jnp.transpose(out_bct, (0, 2, 1, 3))


# speedup vs baseline: 1.0801x; 1.0014x over previous
import jax
import jax.numpy as jnp
from jax import lax
from jax.experimental import pallas as pl
from jax.experimental.pallas import tpu as pltpu
from jax.experimental.pallas import tpu_sc as plsc

NUM_CODEBOOKS = 6
VOCAB = 1000
EMB_DIM = 1024
BATCH = 8
TIME = 1024
TOTAL_ROWS = BATCH * TIME * NUM_CODEBOOKS
TABLE_ROWS = NUM_CODEBOOKS * VOCAB

_NC, _NS, _LANES = 2, 16, 16
_NW = _NC * _NS
_ROWS_PER_W = TOTAL_ROWS // _NW
_CHUNK = 16
_NCHUNK = _ROWS_PER_W // _CHUNK
_NBUF = 6
_AHEAD = 4


def _prep_body(tab_ref, pe_ref, out_ref):
    out_ref[...] = tab_ref[...] + pe_ref[pl.ds(pl.program_id(0), 1), :]


def _make_table_pe(table, pe6):
    blk = VOCAB
    return pl.pallas_call(
        _prep_body,
        grid=(TABLE_ROWS // blk,),
        in_specs=[
            pl.BlockSpec((blk, EMB_DIM), lambda i: (i, 0)),
            pl.BlockSpec((NUM_CODEBOOKS, EMB_DIM), lambda i: (0, 0)),
        ],
        out_specs=pl.BlockSpec((blk, EMB_DIM), lambda i: (i, 0)),
        out_shape=jax.ShapeDtypeStruct((TABLE_ROWS, EMB_DIM), jnp.float32),
    )(table, pe6)


def _sc_pipeline(tok_hbm, tpe_hbm, out_hbm, tok_v, idx_v, bufs, gsems, ssems):
    wid = lax.axis_index("s") * _NC + lax.axis_index("c")
    base = wid * _ROWS_PER_W
    l0 = (wid % 4) * _ROWS_PER_W
    pltpu.sync_copy(tok_hbm.at[pl.ds(pl.multiple_of(base, 8), _ROWS_PER_W)], tok_v)
    for j in range(_ROWS_PER_W // _LANES):
        sl = pl.ds(_LANES * j, _LANES)
        l = lax.iota(jnp.int32, _LANES) + (l0 + _LANES * j)
        idx_v[sl] = tok_v[sl] + lax.shift_right_logical(l, 10) * VOCAB

    def fire_gather(g, rows_v, sem):
        pltpu.async_copy(tpe_hbm.at[idx_v.at[pl.ds(g * _CHUNK, _CHUNK)]], rows_v, sem)

    def wait_gather(g, rows_v, sem):
        pltpu.make_async_copy(
            tpe_hbm.at[idx_v.at[pl.ds(g * _CHUNK, _CHUNK)]], rows_v, sem
        ).wait()

    def out_slice(g):
        return out_hbm.at[pl.ds(pl.multiple_of(base + g * _CHUNK, 8), _CHUNK)]

    def fire_store(g, rows_v, sem):
        pltpu.async_copy(rows_v, out_slice(g), sem)

    def wait_store(g, rows_v, sem):
        pltpu.make_async_copy(rows_v, out_slice(g), sem).wait()

    for a in range(_AHEAD):
        fire_gather(a, bufs[a], gsems[a])

    def ring_body(h, carry):
        for k in range(_NBUF):
            g = _NBUF * h + k
            b2 = (k + _AHEAD) % _NBUF
            wait_gather(g, bufs[k], gsems[k])
            fire_store(g, bufs[k], ssems[k])
            gp = g + _AHEAD - _NBUF

            @pl.when(gp >= 0)
            def _():
                wait_store(gp, bufs[b2], ssems[b2])

            @pl.when(g + _AHEAD < _NCHUNK)
            def _():
                fire_gather(g + _AHEAD, bufs[b2], gsems[b2])

        return carry

    lax.fori_loop(0, _NCHUNK // _NBUF, ring_body, 0)
    for q in range(_NCHUNK + _AHEAD - _NBUF, _NCHUNK):
        wait_store(q, bufs[q % _NBUF], ssems[q % _NBUF])


def _sc_body(tok_hbm, tpe_hbm, out_hbm, tok_v, idx_v,
             rows0, rows1, rows2, rows3, rows4, rows5,
             gsem0, gsem1, gsem2, gsem3, gsem4, gsem5,
             ssem0, ssem1, ssem2, ssem3, ssem4, ssem5):
    _sc_pipeline(
        tok_hbm, tpe_hbm, out_hbm, tok_v, idx_v,
        [rows0, rows1, rows2, rows3, rows4, rows5],
        [gsem0, gsem1, gsem2, gsem3, gsem4, gsem5],
        [ssem0, ssem1, ssem2, ssem3, ssem4, ssem5],
    )


def _sc_gather(tok_flat, table_pe):
    mesh = plsc.VectorSubcoreMesh(core_axis_name="c", subcore_axis_name="s")
    return pl.kernel(
        _sc_body,
        out_type=jax.ShapeDtypeStruct((TOTAL_ROWS, EMB_DIM), jnp.float32),
        mesh=mesh,
        scratch_types=[
            pltpu.VMEM((_ROWS_PER_W,), jnp.int32),
            pltpu.VMEM((_ROWS_PER_W,), jnp.int32),
        ]
        + [pltpu.VMEM((_CHUNK, EMB_DIM), jnp.float32)] * _NBUF
        + [pltpu.SemaphoreType.DMA] * (2 * _NBUF),
    )(tok_flat, table_pe)


def kernel(in_tokens, table, pos_encoding):
    pe6 = pos_encoding.reshape(NUM_CODEBOOKS, EMB_DIM)
    table_pe = _make_table_pe(table, pe6)
    tok_flat = jnp.transpose(in_tokens, (0, 2, 1)).reshape(TOTAL_ROWS)
    out_flat = _sc_gather(tok_flat, table_pe)
    out_bct = out_flat.reshape(BATCH, NUM_CODEBOOKS, TIME, EMB_DIM)
    return jnp.transpose(out_bct, (0, 2, 1, 3))
